# baseline (speedup 1.0000x reference)
import functools
from functools import partial

import jax
import jax.numpy as jnp
from jax import lax
from jax.experimental import pallas as pl
from jax.experimental.pallas import tpu as pltpu
from jax.experimental.pallas import tpu_sc as plsc

N_DOCS = 50000
DOC_DIM = 512
DIM = 256
B = 512
TOPK = 5
KEEP = 4
NCAND = 8
BN = 2000
NB = N_DOCS // BN
NC, NS = 2, 16
NW = NC * NS

_BIG_I = 2**30


@functools.lru_cache(maxsize=None)
def _make_sc_gather(n_rows, d):
    bw = n_rows // NW
    mesh = plsc.VectorSubcoreMesh(
        core_axis_name="c", subcore_axis_name="s",
        num_cores=NC, num_subcores=NS)

    @partial(
        pl.kernel,
        out_type=jax.ShapeDtypeStruct((n_rows, d), jnp.float32),
        mesh=mesh,
        scratch_types=[
            pltpu.VMEM((bw,), jnp.int32),
            pltpu.VMEM((bw, d), jnp.float32),
            pltpu.SemaphoreType.DMA,
        ],
    )
    def gather(table_hbm, idx_hbm, out_hbm, idx_v, rows_v, sem):
        wid = lax.axis_index("s") * NC + lax.axis_index("c")
        base = wid * bw
        pltpu.sync_copy(idx_hbm.at[pl.ds(base, bw)], idx_v)
        pltpu.async_copy(table_hbm.at[idx_v], rows_v, sem).wait()
        pltpu.sync_copy(rows_v, out_hbm.at[pl.ds(base, bw)])

    return gather


_NG = BN // 8
_NH = 4
_UNROLL = 10
_HW = B // _NH


def _main_body(docs_ref, w_ref, targets_ref, tids_ref, ev_ref, q_ref,
               qs, s_scr, rv, ri):
    i = pl.program_id(0)
    w = w_ref[...]

    @pl.when(i == 0)
    def _init():
        q = jnp.dot(targets_ref[...], w, preferred_element_type=jnp.float32)
        qs[...] = -2.0 * q
        q_ref[...] = q
        rv[...] = jnp.full((8 * TOPK, B), jnp.inf, dtype=jnp.float32)
        ri[...] = jnp.zeros((8 * TOPK, B), dtype=jnp.int32)

    de = jnp.dot(docs_ref[...], w, preferred_element_type=jnp.float32)
    dn = jnp.sum(de * de, axis=1, keepdims=True)
    s2 = lax.dot_general(de, qs[...], (((1,), (1,)), ((), ())),
                         preferred_element_type=jnp.float32)
    s_scr[...] = dn + s2

    for h in range(_NH):
        lo = h * _HW
        sl = slice(lo, lo + _HW)
        base = lax.broadcasted_iota(jnp.int32, (8, _HW), 0)
        carry = tuple(rv[8*t:8*t+8, sl] for t in range(TOPK)) + \
                tuple(ri[8*t:8*t+8, sl] for t in range(TOPK))

        def body(g, c, base=base, sl=sl):
            vs, js = list(c[:TOPK]), list(c[TOPK:])
            off = pl.multiple_of(g * 8, 8)
            tv = s_scr[pl.ds(off, 8), sl]
            ti = base + (g * 8 + i * BN)
            nvs, njs = [], []
            for t in range(TOPK):
                swap = tv < vs[t]
                nvs.append(jnp.where(swap, tv, vs[t]))
                njs.append(jnp.where(swap, ti, js[t]))
                if t + 1 < TOPK:
                    tv2 = jnp.where(swap, vs[t], tv)
                    ti = jnp.where(swap, js[t], ti)
                    tv = tv2
            return tuple(nvs) + tuple(njs)

        carry = lax.fori_loop(0, _NG, body, carry, unroll=_UNROLL)
        for t in range(TOPK):
            rv[8*t:8*t+8, sl] = carry[t]
            ri[8*t:8*t+8, sl] = carry[TOPK + t]

    @pl.when(i == NB - 1)
    def _finalize():
        allv = rv[...]
        alli = ri[...]
        ti5 = []
        for _ in range(TOPK):
            m = jnp.min(allv, axis=0, keepdims=True)
            idx = jnp.min(jnp.where(allv == m, alli, _BIG_I), axis=0,
                          keepdims=True)
            ti5.append(idx)
            allv = jnp.where(alli == idx, jnp.inf, allv)

        tid = tids_ref[0:1, :]
        match = [ti5[j] == tid for j in range(TOPK)]
        p = jnp.where(match[0], 0,
            jnp.where(match[1], 1,
            jnp.where(match[2], 2,
            jnp.where(match[3], 3, 4)))).astype(jnp.int32)
        rows = []
        for j in range(KEEP):
            rows.append(jnp.where(jnp.int32(j) < p, ti5[j], ti5[j + 1]))
        ev_ref[0:KEEP, :] = jnp.concatenate(rows, axis=0)
        ev_ref[KEEP:8, :] = jnp.zeros((8 - KEEP, B), dtype=jnp.int32)


def _loss_body(ev_ref, w_ref, q_ref, out_ref):
    e = jnp.dot(ev_ref[...], w_ref[...], preferred_element_type=jnp.float32)
    m = (e[0:B] + e[B:2*B] + e[2*B:3*B] + e[3*B:4*B]) * 0.25 - q_ref[...]
    out_ref[0, 0] = jnp.sum(m * m) / (B * DIM)


def kernel(target_ids, documents, W):
    tids = target_ids.astype(jnp.int32)
    targets = _make_sc_gather(B, DOC_DIM)(documents, tids)
    tids_b = jnp.broadcast_to(tids.reshape(1, B), (8, B))

    ev_ids8, q = pl.pallas_call(
        _main_body,
        grid=(NB,),
        in_specs=[
            pl.BlockSpec((BN, DOC_DIM), lambda i: (i, 0)),
            pl.BlockSpec((DOC_DIM, DIM), lambda i: (0, 0)),
            pl.BlockSpec((B, DOC_DIM), lambda i: (0, 0)),
            pl.BlockSpec((8, B), lambda i: (0, 0)),
        ],
        out_specs=[
            pl.BlockSpec((8, B), lambda i: (0, 0)),
            pl.BlockSpec((B, DIM), lambda i: (0, 0)),
        ],
        out_shape=[
            jax.ShapeDtypeStruct((8, B), jnp.int32),
            jax.ShapeDtypeStruct((B, DIM), jnp.float32),
        ],
        scratch_shapes=[
            pltpu.VMEM((B, DIM), jnp.float32),
            pltpu.VMEM((BN, B), jnp.float32),
            pltpu.VMEM((8 * TOPK, B), jnp.float32),
            pltpu.VMEM((8 * TOPK, B), jnp.int32),
        ],
    )(documents, W, targets, tids_b)

    eidx = ev_ids8[0:KEEP].reshape(KEEP * B)
    evidences = _make_sc_gather(KEEP * B, DOC_DIM)(documents, eidx)

    loss = pl.pallas_call(
        _loss_body,
        in_specs=[
            pl.BlockSpec((KEEP * B, DOC_DIM), lambda: (0, 0)),
            pl.BlockSpec((DOC_DIM, DIM), lambda: (0, 0)),
            pl.BlockSpec((B, DIM), lambda: (0, 0)),
        ],
        out_specs=pl.BlockSpec(memory_space=pltpu.SMEM),
        out_shape=jax.ShapeDtypeStruct((1, 1), jnp.float32),
    )(evidences, W, q)

    return loss[0, 0]


# speedup vs baseline: 1.0475x; 1.0475x over previous
import functools
from functools import partial

import jax
import jax.numpy as jnp
from jax import lax
from jax.experimental import pallas as pl
from jax.experimental.pallas import tpu as pltpu
from jax.experimental.pallas import tpu_sc as plsc

N_DOCS = 50000
DOC_DIM = 512
DIM = 256
B = 512
TOPK = 5
KEEP = 4
NCAND = 8
BN = 2000
NB = N_DOCS // BN
NC, NS = 2, 16
NW = NC * NS

_BIG_I = 2**30


@functools.lru_cache(maxsize=None)
def _make_sc_gather(n_rows, d):
    bw = n_rows // NW
    mesh = plsc.VectorSubcoreMesh(
        core_axis_name="c", subcore_axis_name="s",
        num_cores=NC, num_subcores=NS)

    @partial(
        pl.kernel,
        out_type=jax.ShapeDtypeStruct((n_rows, d), jnp.float32),
        mesh=mesh,
        scratch_types=[
            pltpu.VMEM((bw,), jnp.int32),
            pltpu.VMEM((bw, d), jnp.float32),
            pltpu.SemaphoreType.DMA,
        ],
    )
    def gather(table_hbm, idx_hbm, out_hbm, idx_v, rows_v, sem):
        wid = lax.axis_index("s") * NC + lax.axis_index("c")
        base = wid * bw
        pltpu.sync_copy(idx_hbm.at[pl.ds(base, bw)], idx_v)
        pltpu.async_copy(table_hbm.at[idx_v], rows_v, sem).wait()
        pltpu.sync_copy(rows_v, out_hbm.at[pl.ds(base, bw)])

    return gather


_NG = BN // 8
_NH = 4
_UNROLL = 25
_HW = B // _NH


def _main_body(docs_ref, w_ref, targets_ref, tids_ref, ev_ref, q_ref,
               qs, s_scr, rv, ri):
    i = pl.program_id(0)
    w = w_ref[...]

    @pl.when(i == 0)
    def _init():
        q = jnp.dot(targets_ref[...], w, preferred_element_type=jnp.float32)
        qs[...] = -2.0 * q
        q_ref[...] = q
        rv[...] = jnp.full((8 * TOPK, B), jnp.inf, dtype=jnp.float32)
        ri[...] = jnp.zeros((8 * TOPK, B), dtype=jnp.int32)

    de = jnp.dot(docs_ref[...], w, preferred_element_type=jnp.float32)
    dn = jnp.sum(de * de, axis=1, keepdims=True)
    s2 = lax.dot_general(de, qs[...], (((1,), (1,)), ((), ())),
                         preferred_element_type=jnp.float32)
    s_scr[...] = dn + s2

    for h in range(_NH):
        lo = h * _HW
        sl = slice(lo, lo + _HW)
        base = lax.broadcasted_iota(jnp.int32, (8, _HW), 0)
        carry = tuple(rv[8*t:8*t+8, sl] for t in range(TOPK)) + \
                tuple(ri[8*t:8*t+8, sl] for t in range(TOPK))

        def body(g, c, base=base, sl=sl):
            vs, js = list(c[:TOPK]), list(c[TOPK:])
            off = pl.multiple_of(g * 8, 8)
            tv = s_scr[pl.ds(off, 8), sl]
            ti = base + (g * 8 + i * BN)
            nvs, njs = [], []
            for t in range(TOPK):
                swap = tv < vs[t]
                nvs.append(jnp.where(swap, tv, vs[t]))
                njs.append(jnp.where(swap, ti, js[t]))
                tv2 = jnp.where(swap, vs[t], tv)
                ti = jnp.where(swap, js[t], ti)
                tv = tv2
            return tuple(nvs) + tuple(njs)

        carry = lax.fori_loop(0, _NG, body, carry, unroll=_UNROLL)
        for t in range(TOPK):
            rv[8*t:8*t+8, sl] = carry[t]
            ri[8*t:8*t+8, sl] = carry[TOPK + t]

    @pl.when(i == NB - 1)
    def _finalize():
        allv = rv[...]
        alli = ri[...]
        ti5 = []
        for _ in range(TOPK):
            m = jnp.min(allv, axis=0, keepdims=True)
            idx = jnp.min(jnp.where(allv == m, alli, _BIG_I), axis=0,
                          keepdims=True)
            ti5.append(idx)
            allv = jnp.where(alli == idx, jnp.inf, allv)

        tid = tids_ref[0:1, :]
        match = [ti5[j] == tid for j in range(TOPK)]
        p = jnp.where(match[0], 0,
            jnp.where(match[1], 1,
            jnp.where(match[2], 2,
            jnp.where(match[3], 3, 4)))).astype(jnp.int32)
        rows = []
        for j in range(KEEP):
            rows.append(jnp.where(jnp.int32(j) < p, ti5[j], ti5[j + 1]))
        ev_ref[0:KEEP, :] = jnp.concatenate(rows, axis=0)
        ev_ref[KEEP:8, :] = jnp.zeros((8 - KEEP, B), dtype=jnp.int32)


def _loss_body(ev_ref, w_ref, q_ref, out_ref):
    e = jnp.dot(ev_ref[...], w_ref[...], preferred_element_type=jnp.float32)
    m = (e[0:B] + e[B:2*B] + e[2*B:3*B] + e[3*B:4*B]) * 0.25 - q_ref[...]
    out_ref[0, 0] = jnp.sum(m * m) / (B * DIM)


def kernel(target_ids, documents, W):
    tids = target_ids.astype(jnp.int32)
    targets = _make_sc_gather(B, DOC_DIM)(documents, tids)
    tids_b = jnp.broadcast_to(tids.reshape(1, B), (8, B))

    ev_ids8, q = pl.pallas_call(
        _main_body,
        grid=(NB,),
        in_specs=[
            pl.BlockSpec((BN, DOC_DIM), lambda i: (i, 0)),
            pl.BlockSpec((DOC_DIM, DIM), lambda i: (0, 0)),
            pl.BlockSpec((B, DOC_DIM), lambda i: (0, 0)),
            pl.BlockSpec((8, B), lambda i: (0, 0)),
        ],
        out_specs=[
            pl.BlockSpec((8, B), lambda i: (0, 0)),
            pl.BlockSpec((B, DIM), lambda i: (0, 0)),
        ],
        out_shape=[
            jax.ShapeDtypeStruct((8, B), jnp.int32),
            jax.ShapeDtypeStruct((B, DIM), jnp.float32),
        ],
        scratch_shapes=[
            pltpu.VMEM((B, DIM), jnp.float32),
            pltpu.VMEM((BN, B), jnp.float32),
            pltpu.VMEM((8 * TOPK, B), jnp.float32),
            pltpu.VMEM((8 * TOPK, B), jnp.int32),
        ],
    )(documents, W, targets, tids_b)

    eidx = ev_ids8[0:KEEP].reshape(KEEP * B)
    evidences = _make_sc_gather(KEEP * B, DOC_DIM)(documents, eidx)

    loss = pl.pallas_call(
        _loss_body,
        in_specs=[
            pl.BlockSpec((KEEP * B, DOC_DIM), lambda: (0, 0)),
            pl.BlockSpec((DOC_DIM, DIM), lambda: (0, 0)),
            pl.BlockSpec((B, DIM), lambda: (0, 0)),
        ],
        out_specs=pl.BlockSpec(memory_space=pltpu.SMEM),
        out_shape=jax.ShapeDtypeStruct((1, 1), jnp.float32),
    )(evidences, W, q)

    return loss[0, 0]


# speedup vs baseline: 1.0723x; 1.0237x over previous
import functools
from functools import partial

import jax
import jax.numpy as jnp
from jax import lax
from jax.experimental import pallas as pl
from jax.experimental.pallas import tpu as pltpu
from jax.experimental.pallas import tpu_sc as plsc

N_DOCS = 50000
DOC_DIM = 512
DIM = 256
B = 512
TOPK = 5
KEEP = 4
NCAND = 8
BN = 2000
NB = N_DOCS // BN
NC, NS = 2, 16
NW = NC * NS

_BIG_I = 2**30


@functools.lru_cache(maxsize=None)
def _make_sc_gather(n_rows, d):
    bw = n_rows // NW
    mesh = plsc.VectorSubcoreMesh(
        core_axis_name="c", subcore_axis_name="s",
        num_cores=NC, num_subcores=NS)

    @partial(
        pl.kernel,
        out_type=jax.ShapeDtypeStruct((n_rows, d), jnp.float32),
        mesh=mesh,
        scratch_types=[
            pltpu.VMEM((bw,), jnp.int32),
            pltpu.VMEM((bw, d), jnp.float32),
            pltpu.SemaphoreType.DMA,
        ],
    )
    def gather(table_hbm, idx_hbm, out_hbm, idx_v, rows_v, sem):
        wid = lax.axis_index("s") * NC + lax.axis_index("c")
        base = wid * bw
        pltpu.sync_copy(idx_hbm.at[pl.ds(base, bw)], idx_v)
        pltpu.async_copy(table_hbm.at[idx_v], rows_v, sem).wait()
        pltpu.sync_copy(rows_v, out_hbm.at[pl.ds(base, bw)])

    return gather


_NG = BN // 8
_NH = 2
_UNROLL = 25
_HW = B // _NH


def _main_body(docs_ref, w_ref, targets_ref, tids_ref, ev_ref, q_ref,
               qs, s_scr, rv, ri):
    i = pl.program_id(0)
    w = w_ref[...]

    @pl.when(i == 0)
    def _init():
        q = jnp.dot(targets_ref[...], w, preferred_element_type=jnp.float32)
        qs[...] = -2.0 * q
        q_ref[...] = q
        rv[...] = jnp.full((8 * TOPK, B), jnp.inf, dtype=jnp.float32)
        ri[...] = jnp.zeros((8 * TOPK, B), dtype=jnp.int32)

    de = jnp.dot(docs_ref[...], w, preferred_element_type=jnp.float32)
    dn = jnp.sum(de * de, axis=1, keepdims=True)
    s2 = lax.dot_general(de, qs[...], (((1,), (1,)), ((), ())),
                         preferred_element_type=jnp.float32)
    s_scr[...] = dn + s2

    for h in range(_NH):
        lo = h * _HW
        sl = slice(lo, lo + _HW)
        base = lax.broadcasted_iota(jnp.int32, (8, _HW), 0)
        carry = tuple(rv[8*t:8*t+8, sl] for t in range(TOPK)) + \
                tuple(ri[8*t:8*t+8, sl] for t in range(TOPK))

        def body(g, c, base=base, sl=sl):
            vs, js = list(c[:TOPK]), list(c[TOPK:])
            off = pl.multiple_of(g * 8, 8)
            tv = s_scr[pl.ds(off, 8), sl]
            ti = base + (g * 8 + i * BN)
            nvs, njs = [], []
            for t in range(TOPK):
                swap = tv < vs[t]
                nvs.append(jnp.where(swap, tv, vs[t]))
                njs.append(jnp.where(swap, ti, js[t]))
                tv2 = jnp.where(swap, vs[t], tv)
                ti = jnp.where(swap, js[t], ti)
                tv = tv2
            return tuple(nvs) + tuple(njs)

        carry = lax.fori_loop(0, _NG, body, carry, unroll=_UNROLL)
        for t in range(TOPK):
            rv[8*t:8*t+8, sl] = carry[t]
            ri[8*t:8*t+8, sl] = carry[TOPK + t]

    @pl.when(i == NB - 1)
    def _finalize():
        allv = rv[...]
        alli = ri[...]
        ti5 = []
        for _ in range(TOPK):
            m = jnp.min(allv, axis=0, keepdims=True)
            idx = jnp.min(jnp.where(allv == m, alli, _BIG_I), axis=0,
                          keepdims=True)
            ti5.append(idx)
            allv = jnp.where(alli == idx, jnp.inf, allv)

        tid = tids_ref[0:1, :]
        match = [ti5[j] == tid for j in range(TOPK)]
        p = jnp.where(match[0], 0,
            jnp.where(match[1], 1,
            jnp.where(match[2], 2,
            jnp.where(match[3], 3, 4)))).astype(jnp.int32)
        rows = []
        for j in range(KEEP):
            rows.append(jnp.where(jnp.int32(j) < p, ti5[j], ti5[j + 1]))
        ev_ref[0:KEEP, :] = jnp.concatenate(rows, axis=0)
        ev_ref[KEEP:8, :] = jnp.zeros((8 - KEEP, B), dtype=jnp.int32)


def _loss_body(ev_ref, w_ref, q_ref, out_ref):
    e = jnp.dot(ev_ref[...], w_ref[...], preferred_element_type=jnp.float32)
    m = (e[0:B] + e[B:2*B] + e[2*B:3*B] + e[3*B:4*B]) * 0.25 - q_ref[...]
    out_ref[0, 0] = jnp.sum(m * m) / (B * DIM)


def kernel(target_ids, documents, W):
    tids = target_ids.astype(jnp.int32)
    targets = _make_sc_gather(B, DOC_DIM)(documents, tids)
    tids_b = jnp.broadcast_to(tids.reshape(1, B), (8, B))

    ev_ids8, q = pl.pallas_call(
        _main_body,
        grid=(NB,),
        in_specs=[
            pl.BlockSpec((BN, DOC_DIM), lambda i: (i, 0)),
            pl.BlockSpec((DOC_DIM, DIM), lambda i: (0, 0)),
            pl.BlockSpec((B, DOC_DIM), lambda i: (0, 0)),
            pl.BlockSpec((8, B), lambda i: (0, 0)),
        ],
        out_specs=[
            pl.BlockSpec((8, B), lambda i: (0, 0)),
            pl.BlockSpec((B, DIM), lambda i: (0, 0)),
        ],
        out_shape=[
            jax.ShapeDtypeStruct((8, B), jnp.int32),
            jax.ShapeDtypeStruct((B, DIM), jnp.float32),
        ],
        scratch_shapes=[
            pltpu.VMEM((B, DIM), jnp.float32),
            pltpu.VMEM((BN, B), jnp.float32),
            pltpu.VMEM((8 * TOPK, B), jnp.float32),
            pltpu.VMEM((8 * TOPK, B), jnp.int32),
        ],
    )(documents, W, targets, tids_b)

    eidx = ev_ids8[0:KEEP].reshape(KEEP * B)
    evidences = _make_sc_gather(KEEP * B, DOC_DIM)(documents, eidx)

    loss = pl.pallas_call(
        _loss_body,
        in_specs=[
            pl.BlockSpec((KEEP * B, DOC_DIM), lambda: (0, 0)),
            pl.BlockSpec((DOC_DIM, DIM), lambda: (0, 0)),
            pl.BlockSpec((B, DIM), lambda: (0, 0)),
        ],
        out_specs=pl.BlockSpec(memory_space=pltpu.SMEM),
        out_shape=jax.ShapeDtypeStruct((1, 1), jnp.float32),
    )(evidences, W, q)

    return loss[0, 0]


# speedup vs baseline: 1.0965x; 1.0226x over previous
import functools
from functools import partial

import jax
import jax.numpy as jnp
from jax import lax
from jax.experimental import pallas as pl
from jax.experimental.pallas import tpu as pltpu
from jax.experimental.pallas import tpu_sc as plsc

N_DOCS = 50000
DOC_DIM = 512
DIM = 256
B = 512
TOPK = 5
KEEP = 4
NCAND = 8
BN = 2000
NB = N_DOCS // BN
NC, NS = 2, 16
NW = NC * NS

_BIG_I = 2**30


@functools.lru_cache(maxsize=None)
def _make_sc_gather(n_rows, d):
    bw = n_rows // NW
    mesh = plsc.VectorSubcoreMesh(
        core_axis_name="c", subcore_axis_name="s",
        num_cores=NC, num_subcores=NS)

    @partial(
        pl.kernel,
        out_type=jax.ShapeDtypeStruct((n_rows, d), jnp.float32),
        mesh=mesh,
        scratch_types=[
            pltpu.VMEM((bw,), jnp.int32),
            pltpu.VMEM((bw, d), jnp.float32),
            pltpu.SemaphoreType.DMA,
        ],
    )
    def gather(table_hbm, idx_hbm, out_hbm, idx_v, rows_v, sem):
        wid = lax.axis_index("s") * NC + lax.axis_index("c")
        base = wid * bw
        pltpu.sync_copy(idx_hbm.at[pl.ds(base, bw)], idx_v)
        pltpu.async_copy(table_hbm.at[idx_v], rows_v, sem).wait()
        pltpu.sync_copy(rows_v, out_hbm.at[pl.ds(base, bw)])

    return gather


_NG = BN // 8
_NH = 1
_UNROLL = 25
_HW = B // _NH


def _main_body(docs_ref, w_ref, targets_ref, tids_ref, ev_ref, q_ref,
               qs, s_scr, rv, ri):
    i = pl.program_id(0)
    w = w_ref[...]

    @pl.when(i == 0)
    def _init():
        q = jnp.dot(targets_ref[...], w, preferred_element_type=jnp.float32)
        qs[...] = -2.0 * q
        q_ref[...] = q
        rv[...] = jnp.full((8 * TOPK, B), jnp.inf, dtype=jnp.float32)
        ri[...] = jnp.zeros((8 * TOPK, B), dtype=jnp.int32)

    de = jnp.dot(docs_ref[...], w, preferred_element_type=jnp.float32)
    dn = jnp.sum(de * de, axis=1, keepdims=True)
    s2 = lax.dot_general(de, qs[...], (((1,), (1,)), ((), ())),
                         preferred_element_type=jnp.float32)
    s_scr[...] = dn + s2

    for h in range(_NH):
        lo = h * _HW
        sl = slice(lo, lo + _HW)
        base = lax.broadcasted_iota(jnp.int32, (8, _HW), 0)
        carry = tuple(rv[8*t:8*t+8, sl] for t in range(TOPK)) + \
                tuple(ri[8*t:8*t+8, sl] for t in range(TOPK))

        def body(g, c, base=base, sl=sl):
            vs, js = list(c[:TOPK]), list(c[TOPK:])
            off = pl.multiple_of(g * 8, 8)
            tv = s_scr[pl.ds(off, 8), sl]
            ti = base + (g * 8 + i * BN)
            nvs, njs = [], []
            for t in range(TOPK):
                swap = tv < vs[t]
                nvs.append(jnp.where(swap, tv, vs[t]))
                njs.append(jnp.where(swap, ti, js[t]))
                tv2 = jnp.where(swap, vs[t], tv)
                ti = jnp.where(swap, js[t], ti)
                tv = tv2
            return tuple(nvs) + tuple(njs)

        carry = lax.fori_loop(0, _NG, body, carry, unroll=_UNROLL)
        for t in range(TOPK):
            rv[8*t:8*t+8, sl] = carry[t]
            ri[8*t:8*t+8, sl] = carry[TOPK + t]

    @pl.when(i == NB - 1)
    def _finalize():
        allv = rv[...]
        alli = ri[...]
        ti5 = []
        for _ in range(TOPK):
            m = jnp.min(allv, axis=0, keepdims=True)
            idx = jnp.min(jnp.where(allv == m, alli, _BIG_I), axis=0,
                          keepdims=True)
            ti5.append(idx)
            allv = jnp.where(alli == idx, jnp.inf, allv)

        tid = tids_ref[0:1, :]
        match = [ti5[j] == tid for j in range(TOPK)]
        p = jnp.where(match[0], 0,
            jnp.where(match[1], 1,
            jnp.where(match[2], 2,
            jnp.where(match[3], 3, 4)))).astype(jnp.int32)
        rows = []
        for j in range(KEEP):
            rows.append(jnp.where(jnp.int32(j) < p, ti5[j], ti5[j + 1]))
        ev_ref[0:KEEP, :] = jnp.concatenate(rows, axis=0)
        ev_ref[KEEP:8, :] = jnp.zeros((8 - KEEP, B), dtype=jnp.int32)


def _loss_body(ev_ref, w_ref, q_ref, out_ref):
    e = jnp.dot(ev_ref[...], w_ref[...], preferred_element_type=jnp.float32)
    m = (e[0:B] + e[B:2*B] + e[2*B:3*B] + e[3*B:4*B]) * 0.25 - q_ref[...]
    out_ref[0, 0] = jnp.sum(m * m) / (B * DIM)


def kernel(target_ids, documents, W):
    tids = target_ids.astype(jnp.int32)
    targets = _make_sc_gather(B, DOC_DIM)(documents, tids)
    tids_b = jnp.broadcast_to(tids.reshape(1, B), (8, B))

    ev_ids8, q = pl.pallas_call(
        _main_body,
        grid=(NB,),
        in_specs=[
            pl.BlockSpec((BN, DOC_DIM), lambda i: (i, 0)),
            pl.BlockSpec((DOC_DIM, DIM), lambda i: (0, 0)),
            pl.BlockSpec((B, DOC_DIM), lambda i: (0, 0)),
            pl.BlockSpec((8, B), lambda i: (0, 0)),
        ],
        out_specs=[
            pl.BlockSpec((8, B), lambda i: (0, 0)),
            pl.BlockSpec((B, DIM), lambda i: (0, 0)),
        ],
        out_shape=[
            jax.ShapeDtypeStruct((8, B), jnp.int32),
            jax.ShapeDtypeStruct((B, DIM), jnp.float32),
        ],
        scratch_shapes=[
            pltpu.VMEM((B, DIM), jnp.float32),
            pltpu.VMEM((BN, B), jnp.float32),
            pltpu.VMEM((8 * TOPK, B), jnp.float32),
            pltpu.VMEM((8 * TOPK, B), jnp.int32),
        ],
    )(documents, W, targets, tids_b)

    eidx = ev_ids8[0:KEEP].reshape(KEEP * B)
    evidences = _make_sc_gather(KEEP * B, DOC_DIM)(documents, eidx)

    loss = pl.pallas_call(
        _loss_body,
        in_specs=[
            pl.BlockSpec((KEEP * B, DOC_DIM), lambda: (0, 0)),
            pl.BlockSpec((DOC_DIM, DIM), lambda: (0, 0)),
            pl.BlockSpec((B, DIM), lambda: (0, 0)),
        ],
        out_specs=pl.BlockSpec(memory_space=pltpu.SMEM),
        out_shape=jax.ShapeDtypeStruct((1, 1), jnp.float32),
    )(evidences, W, q)

    return loss[0, 0]


# speedup vs baseline: 1.1000x; 1.0031x over previous
import functools
from functools import partial

import jax
import jax.numpy as jnp
from jax import lax
from jax.experimental import pallas as pl
from jax.experimental.pallas import tpu as pltpu
from jax.experimental.pallas import tpu_sc as plsc

N_DOCS = 50000
DOC_DIM = 512
DIM = 256
B = 512
TOPK = 5
KEEP = 4
NCAND = 8
BN = 2000
NB = N_DOCS // BN
NC, NS = 2, 16
NW = NC * NS

_BIG_I = 2**30


@functools.lru_cache(maxsize=None)
def _make_sc_gather(n_rows, d):
    bw = n_rows // NW
    mesh = plsc.VectorSubcoreMesh(
        core_axis_name="c", subcore_axis_name="s",
        num_cores=NC, num_subcores=NS)

    @partial(
        pl.kernel,
        out_type=jax.ShapeDtypeStruct((n_rows, d), jnp.float32),
        mesh=mesh,
        scratch_types=[
            pltpu.VMEM((bw,), jnp.int32),
            pltpu.VMEM((bw, d), jnp.float32),
            pltpu.SemaphoreType.DMA,
        ],
    )
    def gather(table_hbm, idx_hbm, out_hbm, idx_v, rows_v, sem):
        wid = lax.axis_index("s") * NC + lax.axis_index("c")
        base = wid * bw
        pltpu.sync_copy(idx_hbm.at[pl.ds(base, bw)], idx_v)
        pltpu.async_copy(table_hbm.at[idx_v], rows_v, sem).wait()
        pltpu.sync_copy(rows_v, out_hbm.at[pl.ds(base, bw)])

    return gather


_NG = BN // 8
_NH = 1
_UNROLL = 50
_HW = B // _NH


def _main_body(docs_ref, w_ref, targets_ref, tids_ref, ev_ref, q_ref,
               qs, s_scr, rv, ri):
    i = pl.program_id(0)
    w = w_ref[...]

    @pl.when(i == 0)
    def _init():
        q = jnp.dot(targets_ref[...], w, preferred_element_type=jnp.float32)
        qs[...] = -2.0 * q
        q_ref[...] = q
        rv[...] = jnp.full((8 * TOPK, B), jnp.inf, dtype=jnp.float32)
        ri[...] = jnp.zeros((8 * TOPK, B), dtype=jnp.int32)

    de = jnp.dot(docs_ref[...], w, preferred_element_type=jnp.float32)
    dn = jnp.sum(de * de, axis=1, keepdims=True)
    s2 = lax.dot_general(de, qs[...], (((1,), (1,)), ((), ())),
                         preferred_element_type=jnp.float32)
    s_scr[...] = dn + s2

    for h in range(_NH):
        lo = h * _HW
        sl = slice(lo, lo + _HW)
        base = lax.broadcasted_iota(jnp.int32, (8, _HW), 0)
        carry = tuple(rv[8*t:8*t+8, sl] for t in range(TOPK)) + \
                tuple(ri[8*t:8*t+8, sl] for t in range(TOPK))

        def body(g, c, base=base, sl=sl):
            vs, js = list(c[:TOPK]), list(c[TOPK:])
            off = pl.multiple_of(g * 8, 8)
            tv = s_scr[pl.ds(off, 8), sl]
            ti = base + (g * 8 + i * BN)
            nvs, njs = [], []
            for t in range(TOPK):
                swap = tv < vs[t]
                nvs.append(jnp.where(swap, tv, vs[t]))
                njs.append(jnp.where(swap, ti, js[t]))
                tv2 = jnp.where(swap, vs[t], tv)
                ti = jnp.where(swap, js[t], ti)
                tv = tv2
            return tuple(nvs) + tuple(njs)

        carry = lax.fori_loop(0, _NG, body, carry, unroll=_UNROLL)
        for t in range(TOPK):
            rv[8*t:8*t+8, sl] = carry[t]
            ri[8*t:8*t+8, sl] = carry[TOPK + t]

    @pl.when(i == NB - 1)
    def _finalize():
        allv = rv[...]
        alli = ri[...]
        ti5 = []
        for _ in range(TOPK):
            m = jnp.min(allv, axis=0, keepdims=True)
            idx = jnp.min(jnp.where(allv == m, alli, _BIG_I), axis=0,
                          keepdims=True)
            ti5.append(idx)
            allv = jnp.where(alli == idx, jnp.inf, allv)

        tid = tids_ref[0:1, :]
        match = [ti5[j] == tid for j in range(TOPK)]
        p = jnp.where(match[0], 0,
            jnp.where(match[1], 1,
            jnp.where(match[2], 2,
            jnp.where(match[3], 3, 4)))).astype(jnp.int32)
        rows = []
        for j in range(KEEP):
            rows.append(jnp.where(jnp.int32(j) < p, ti5[j], ti5[j + 1]))
        ev_ref[0:KEEP, :] = jnp.concatenate(rows, axis=0)
        ev_ref[KEEP:8, :] = jnp.zeros((8 - KEEP, B), dtype=jnp.int32)


def _loss_body(ev_ref, w_ref, q_ref, out_ref):
    e = jnp.dot(ev_ref[...], w_ref[...], preferred_element_type=jnp.float32)
    m = (e[0:B] + e[B:2*B] + e[2*B:3*B] + e[3*B:4*B]) * 0.25 - q_ref[...]
    out_ref[0, 0] = jnp.sum(m * m) / (B * DIM)


def kernel(target_ids, documents, W):
    tids = target_ids.astype(jnp.int32)
    targets = _make_sc_gather(B, DOC_DIM)(documents, tids)
    tids_b = jnp.broadcast_to(tids.reshape(1, B), (8, B))

    ev_ids8, q = pl.pallas_call(
        _main_body,
        grid=(NB,),
        in_specs=[
            pl.BlockSpec((BN, DOC_DIM), lambda i: (i, 0)),
            pl.BlockSpec((DOC_DIM, DIM), lambda i: (0, 0)),
            pl.BlockSpec((B, DOC_DIM), lambda i: (0, 0)),
            pl.BlockSpec((8, B), lambda i: (0, 0)),
        ],
        out_specs=[
            pl.BlockSpec((8, B), lambda i: (0, 0)),
            pl.BlockSpec((B, DIM), lambda i: (0, 0)),
        ],
        out_shape=[
            jax.ShapeDtypeStruct((8, B), jnp.int32),
            jax.ShapeDtypeStruct((B, DIM), jnp.float32),
        ],
        scratch_shapes=[
            pltpu.VMEM((B, DIM), jnp.float32),
            pltpu.VMEM((BN, B), jnp.float32),
            pltpu.VMEM((8 * TOPK, B), jnp.float32),
            pltpu.VMEM((8 * TOPK, B), jnp.int32),
        ],
    )(documents, W, targets, tids_b)

    eidx = ev_ids8[0:KEEP].reshape(KEEP * B)
    evidences = _make_sc_gather(KEEP * B, DOC_DIM)(documents, eidx)

    loss = pl.pallas_call(
        _loss_body,
        in_specs=[
            pl.BlockSpec((KEEP * B, DOC_DIM), lambda: (0, 0)),
            pl.BlockSpec((DOC_DIM, DIM), lambda: (0, 0)),
            pl.BlockSpec((B, DIM), lambda: (0, 0)),
        ],
        out_specs=pl.BlockSpec(memory_space=pltpu.SMEM),
        out_shape=jax.ShapeDtypeStruct((1, 1), jnp.float32),
    )(evidences, W, q)

    return loss[0, 0]
